# Initial kernel scaffold; baseline (speedup 1.0000x reference)
import jax
import jax.numpy as jnp
from jax.experimental import pallas as pl


def kernel(x, W1, Wself1, W2, Wself2, rel_emb, neigh_idx, neigh_rel, triples):
    raise NotImplementedError("write your pallas kernel here")


# speedup vs baseline: 1.4337x; 1.4337x over previous
import functools

import jax
import jax.numpy as jnp
from jax import lax
from jax.experimental import pallas as pl
from jax.experimental.pallas import tpu as pltpu
from jax.experimental.pallas import tpu_sc as plsc

N = 10000
R = 16
S = 10
D = 128
B = 320000

NC = 2
NS = 16
NW = NC * NS

CH = 64
CPW = 5
NP = NW * CPW * CH
GN = CH * S
GSUB = GN // 128

CT = 80
TPW = B // NW
DCHUNKS = TPW // CT

_mesh = lambda: plsc.VectorSubcoreMesh(
    core_axis_name="c", subcore_axis_name="s", num_cores=NC, num_subcores=NS)

_sc_params = lambda: pltpu.CompilerParams(needs_layout_passes=False)


def _wid():
    return lax.axis_index("s") * NC + lax.axis_index("c")


def _xw_body(x_ref, wn_ref, ws_ref, on_ref, os_ref):
    x = x_ref[...]
    on_ref[...] = jnp.dot(x, wn_ref[...], preferred_element_type=jnp.float32)
    os_ref[...] = jnp.dot(x, ws_ref[...], preferred_element_type=jnp.float32)


def _transform(h_pad, w_neigh, w_self):
    BN = 512
    return pl.pallas_call(
        _xw_body,
        grid=(NP // BN,),
        in_specs=[
            pl.BlockSpec((BN, D), lambda i: (i, 0)),
            pl.BlockSpec((D, R * D), lambda i: (0, 0)),
            pl.BlockSpec((D, D), lambda i: (0, 0)),
        ],
        out_specs=[
            pl.BlockSpec((BN, R * D), lambda i: (i, 0)),
            pl.BlockSpec((BN, D), lambda i: (i, 0)),
        ],
        out_shape=[
            jax.ShapeDtypeStruct((NP, R * D), jnp.float32),
            jax.ShapeDtypeStruct((NP, D), jnp.float32),
        ],
    )(h_pad, w_neigh, w_self)


def _agg_body(xwn_hbm, xws_hbm, nidx_hbm, out_hbm, *rest):
    nidx_bufs = rest[:GSUB]
    rows_v, self_v, out_v, sem = rest[GSUB:]
    wid = _wid()

    def chunk_body(c, _):
        g = wid * CPW + c
        for j in range(GSUB):
            pltpu.sync_copy(nidx_hbm.at[pl.ds(g * GN + j * 128, 128)],
                            nidx_bufs[j])
        pltpu.sync_copy(xws_hbm.at[pl.ds(g * CH, CH)], self_v)
        descs = []
        for j in range(GSUB):
            descs.append(pltpu.async_copy(
                xwn_hbm.at[nidx_bufs[j]],
                rows_v.at[pl.ds(j * 128, 128)], sem))
        for d in descs:
            d.wait()

        def node_body(i, _):
            base = i * S
            for dk in range(D // 16):
                sl = pl.ds(dk * 16, 16)
                acc = rows_v[base, sl]
                for s in range(1, S):
                    acc = acc + rows_v[base + s, sl]
                h = jnp.maximum(self_v[i, sl] + acc * (1.0 / S), 0.0)
                out_v[i, sl] = h
            return 0

        lax.fori_loop(0, CH, node_body, 0)
        pltpu.sync_copy(out_v, out_hbm.at[pl.ds(g * CH, CH)])
        return 0

    lax.fori_loop(0, CPW, chunk_body, 0)


def _aggregate(xwn_flat, xws, nidx_flat):
    k = functools.partial(
        pl.kernel,
        out_type=jax.ShapeDtypeStruct((NP, D), jnp.float32),
        mesh=_mesh(),
        compiler_params=_sc_params(),
        scratch_types=(
            [pltpu.VMEM((128,), jnp.int32)] * GSUB + [
                pltpu.VMEM((GN, D), jnp.float32),
                pltpu.VMEM((CH, D), jnp.float32),
                pltpu.VMEM((CH, D), jnp.float32),
                pltpu.SemaphoreType.DMA,
            ]),
    )(_agg_body)
    return k(xwn_flat, xws, nidx_flat)


def _dm_body(h2_hbm, rel_hbm, si_hbm, oi_hbm, ri_hbm, out_hbm,
             si_v, oi_v, ri_v, sr_v, or_v, rr_v, out_v, sem):
    wid = _wid()

    def chunk_body(c, _):
        base = wid * TPW + c * CT
        pltpu.sync_copy(si_hbm.at[pl.ds(base, CT)], si_v)
        pltpu.sync_copy(oi_hbm.at[pl.ds(base, CT)], oi_v)
        pltpu.sync_copy(ri_hbm.at[pl.ds(base, CT)], ri_v)
        d1 = pltpu.async_copy(h2_hbm.at[si_v], sr_v, sem)
        d2 = pltpu.async_copy(h2_hbm.at[oi_v], or_v, sem)
        d3 = pltpu.async_copy(rel_hbm.at[ri_v], rr_v, sem)
        d1.wait()
        d2.wait()
        d3.wait()
        lanes = lax.broadcasted_iota(jnp.int32, (16,), 0)

        def grp_body(j, _):
            scores = jnp.zeros((16,), jnp.float32)
            for t in range(16):
                i = j * 16 + t
                acc = jnp.zeros((16,), jnp.float32)
                for dk in range(D // 16):
                    sl = pl.ds(dk * 16, 16)
                    acc = acc + sr_v[i, sl] * or_v[i, sl] * rr_v[i, sl]
                sc = jnp.sum(acc)
                scores = jnp.where(lanes == t, sc, scores)
            out_v[pl.ds(j * 16, 16)] = scores
            return 0

        lax.fori_loop(0, CT // 16, grp_body, 0)
        pltpu.sync_copy(out_v, out_hbm.at[pl.ds(base, CT)])
        return 0

    lax.fori_loop(0, DCHUNKS, chunk_body, 0)


def _distmult(h2_pad, rel_emb, subj, obj, rel):
    k = functools.partial(
        pl.kernel,
        out_type=jax.ShapeDtypeStruct((B,), jnp.float32),
        mesh=_mesh(),
        compiler_params=_sc_params(),
        scratch_types=[
            pltpu.VMEM((CT,), jnp.int32),
            pltpu.VMEM((CT,), jnp.int32),
            pltpu.VMEM((CT,), jnp.int32),
            pltpu.VMEM((CT, D), jnp.float32),
            pltpu.VMEM((CT, D), jnp.float32),
            pltpu.VMEM((CT, D), jnp.float32),
            pltpu.VMEM((CT,), jnp.float32),
            pltpu.SemaphoreType.DMA,
        ],
    )(_dm_body)
    return k(h2_pad, rel_emb, subj, obj, rel)


def _layer(h_pad, W, Wself, nidx_flat):
    w_neigh = jnp.transpose(W, (1, 0, 2)).reshape(D, R * D)
    xwn, xws = _transform(h_pad, w_neigh, Wself)
    xwn_flat = xwn.reshape(NP * R, D)
    return _aggregate(xwn_flat, xws, nidx_flat)


def kernel(x, W1, Wself1, W2, Wself2, rel_emb, neigh_idx, neigh_rel, triples):
    flat = (neigh_idx.astype(jnp.int32) * R + neigh_rel.astype(jnp.int32))
    flat = jnp.pad(flat, ((0, NP - N), (0, 0)))
    nidx_flat = flat.reshape(NP * S)

    x_pad = jnp.pad(x, ((0, NP - N), (0, 0)))
    h1 = _layer(x_pad, W1, Wself1, nidx_flat)
    h2 = _layer(h1, W2, Wself2, nidx_flat)

    subj = triples[:, 0].astype(jnp.int32)
    obj = triples[:, 1].astype(jnp.int32)
    rel = triples[:, 2].astype(jnp.int32)
    scores = _distmult(h2, rel_emb, subj, obj, rel)
    return scores.reshape(B, 1)


# speedup vs baseline: 2.4824x; 1.7315x over previous
import functools

import jax
import jax.numpy as jnp
from jax import lax
from jax.experimental import pallas as pl
from jax.experimental.pallas import tpu as pltpu
from jax.experimental.pallas import tpu_sc as plsc

N = 10000
R = 16
S = 10
D = 128
B = 320000

NC = 2
NS = 16
NW = NC * NS

CH = 64
CPW = 5
NP = NW * CPW * CH
GN = CH * S
GSUB = GN // 128

CT = 80
TPW = B // NW
DCHUNKS = TPW // CT

_mesh = lambda: plsc.VectorSubcoreMesh(
    core_axis_name="c", subcore_axis_name="s", num_cores=NC, num_subcores=NS)

_sc_params = lambda: pltpu.CompilerParams(needs_layout_passes=False, use_tc_tiling_on_sc=False)


def _wid():
    return lax.axis_index("s") * NC + lax.axis_index("c")


def _xw_body(x_ref, wn_ref, ws_ref, on_ref, os_ref):
    x = x_ref[...]
    on_ref[...] = jnp.dot(x, wn_ref[...], preferred_element_type=jnp.float32)
    os_ref[...] = jnp.dot(x, ws_ref[...], preferred_element_type=jnp.float32)


def _transform(h_pad, w_neigh, w_self):
    BN = 512
    return pl.pallas_call(
        _xw_body,
        grid=(NP // BN,),
        in_specs=[
            pl.BlockSpec((BN, D), lambda i: (i, 0)),
            pl.BlockSpec((D, R * D), lambda i: (0, 0)),
            pl.BlockSpec((D, D), lambda i: (0, 0)),
        ],
        out_specs=[
            pl.BlockSpec((BN, R * D), lambda i: (i, 0)),
            pl.BlockSpec((BN, D), lambda i: (i, 0)),
        ],
        out_shape=[
            jax.ShapeDtypeStruct((NP, R * D), jnp.float32),
            jax.ShapeDtypeStruct((NP, D), jnp.float32),
        ],
    )(h_pad, w_neigh, w_self)


def _agg_body(xwn_hbm, xws_hbm, nidx_hbm, out_hbm, *rest):
    nidx_bufs = rest[:GSUB]
    rows_v, self_v, out_v, sem = rest[GSUB:]
    wid = _wid()

    def chunk_body(c, _):
        g = wid * CPW + c
        for j in range(GSUB):
            pltpu.sync_copy(nidx_hbm.at[pl.ds(g * GN + j * 128, 128)],
                            nidx_bufs[j])
        pltpu.sync_copy(xws_hbm.at[pl.ds(g * CH, CH)], self_v)
        descs = []
        for j in range(GSUB):
            descs.append(pltpu.async_copy(
                xwn_hbm.at[nidx_bufs[j]],
                rows_v.at[pl.ds(j * 128, 128)], sem))
        for d in descs:
            d.wait()

        def node_body(i, _):
            base = i * S
            for dk in range(D // 16):
                sl = pl.ds(dk * 16, 16)
                acc = rows_v[base, sl]
                for s in range(1, S):
                    acc = acc + rows_v[base + s, sl]
                h = jnp.maximum(self_v[i, sl] + acc * (1.0 / S), 0.0)
                out_v[i, sl] = h
            return 0

        lax.fori_loop(0, CH, node_body, 0)
        pltpu.sync_copy(out_v, out_hbm.at[pl.ds(g * CH, CH)])
        return 0

    lax.fori_loop(0, CPW, chunk_body, 0)


def _aggregate(xwn_flat, xws, nidx_flat):
    k = functools.partial(
        pl.kernel,
        out_type=jax.ShapeDtypeStruct((NP, D), jnp.float32),
        mesh=_mesh(),
        compiler_params=_sc_params(),
        scratch_types=(
            [pltpu.VMEM((128,), jnp.int32)] * GSUB + [
                pltpu.VMEM((GN, D), jnp.float32),
                pltpu.VMEM((CH, D), jnp.float32),
                pltpu.VMEM((CH, D), jnp.float32),
                pltpu.SemaphoreType.DMA,
            ]),
    )(_agg_body)
    return k(xwn_flat, xws, nidx_flat)


NSLOT = 5


def _dm_body(h2_hbm, rel_hbm, si_hbm, oi_hbm, ri_hbm, out_hbm, *rest):
    si_v = rest[0:NSLOT]
    oi_v = rest[NSLOT:2 * NSLOT]
    sr_v = rest[2 * NSLOT:3 * NSLOT]
    or_v = rest[3 * NSLOT:4 * NSLOT]
    out_v = rest[4 * NSLOT:5 * NSLOT]
    ri_v = rest[5 * NSLOT:6 * NSLOT]
    sems = rest[6 * NSLOT:7 * NSLOT]
    rtab_v = rest[7 * NSLOT]
    wid = _wid()
    lanes = lax.broadcasted_iota(jnp.int32, (16,), 0)

    pltpu.sync_copy(rel_hbm, rtab_v)

    def fire(slot, sc):
        base = wid * TPW + sc * CT
        pltpu.sync_copy(si_hbm.at[pl.ds(base, CT)], si_v[slot])
        pltpu.sync_copy(oi_hbm.at[pl.ds(base, CT)], oi_v[slot])
        pltpu.sync_copy(ri_hbm.at[pl.ds(base, CT)], ri_v[slot])
        pltpu.async_copy(h2_hbm.at[si_v[slot]], sr_v[slot], sems[slot])
        pltpu.async_copy(h2_hbm.at[oi_v[slot]], or_v[slot], sems[slot])

    for k in range(NSLOT - 1):
        fire(k, k)

    def iter_body(it, _):
        for k in range(NSLOT):
            sc = it * NSLOT + k
            kn = (k + NSLOT - 1) % NSLOT

            @pl.when(sc + NSLOT - 1 < DCHUNKS)
            def _():
                fire(kn, sc + NSLOT - 1)

            pltpu.make_async_copy(h2_hbm.at[si_v[k]], sr_v[k], sems[k]).wait()
            pltpu.make_async_copy(h2_hbm.at[oi_v[k]], or_v[k], sems[k]).wait()

            def grp_body(g, _):
                scores = jnp.zeros((16,), jnp.float32)
                rel_vec = ri_v[k][pl.ds(g * 16, 16)]
                for t in range(16):
                    i = g * 16 + t
                    ri = rel_vec[t]
                    acc = jnp.zeros((16,), jnp.float32)
                    for q in range(D // 32):
                        sq = plsc.bitcast(
                            sr_v[k][i, pl.ds(q * 16, 16)], jnp.bfloat16)
                        oq = plsc.bitcast(
                            or_v[k][i, pl.ds(q * 16, 16)], jnp.bfloat16)
                        a0, a1 = plsc.unpack(
                            sq, format=plsc.PackFormat.INTERLEAVED)
                        b0, b1 = plsc.unpack(
                            oq, format=plsc.PackFormat.INTERLEAVED)
                        c0 = rtab_v[ri, pl.ds(q * 32, 16)]
                        c1 = rtab_v[ri, pl.ds(q * 32 + 16, 16)]
                        acc = acc + a0 * b0 * c0
                        acc = acc + a1 * b1 * c1
                    scv = jnp.sum(acc)
                    scores = jnp.where(lanes == t, scv, scores)
                out_v[k][pl.ds(g * 16, 16)] = scores
                return 0

            lax.fori_loop(0, CT // 16, grp_body, 0)
            pltpu.sync_copy(out_v[k],
                            out_hbm.at[pl.ds(wid * TPW + sc * CT, CT)])
        return 0

    lax.fori_loop(0, DCHUNKS // NSLOT, iter_body, 0)


def _distmult(h2_bf, rel_perm, subj, obj, rel):
    k = functools.partial(
        pl.kernel,
        out_type=jax.ShapeDtypeStruct((B,), jnp.float32),
        mesh=_mesh(),
        compiler_params=_sc_params(),
        scratch_types=(
            [pltpu.VMEM((CT,), jnp.int32)] * NSLOT
            + [pltpu.VMEM((CT,), jnp.int32)] * NSLOT
            + [pltpu.VMEM((CT, D // 2), jnp.int32)] * NSLOT
            + [pltpu.VMEM((CT, D // 2), jnp.int32)] * NSLOT
            + [pltpu.VMEM((CT,), jnp.float32)] * NSLOT
            + [pltpu.VMEM((CT,), jnp.int32)] * NSLOT
            + [pltpu.SemaphoreType.DMA] * NSLOT
            + [pltpu.VMEM((R, D), jnp.float32)]
        ),
    )(_dm_body)
    return k(h2_bf, rel_perm, subj, obj, rel)


def _layer(h_pad, W, Wself, nidx_flat):
    w_neigh = jnp.transpose(W, (1, 0, 2)).reshape(D, R * D)
    xwn, xws = _transform(h_pad, w_neigh, Wself)
    xwn_flat = xwn.reshape(NP * R, D)
    return _aggregate(xwn_flat, xws, nidx_flat)


def kernel(x, W1, Wself1, W2, Wself2, rel_emb, neigh_idx, neigh_rel, triples):
    flat = (neigh_idx.astype(jnp.int32) * R + neigh_rel.astype(jnp.int32))
    flat = jnp.pad(flat, ((0, NP - N), (0, 0)))
    nidx_flat = flat.reshape(NP * S)

    x_pad = jnp.pad(x, ((0, NP - N), (0, 0)))
    h1 = _layer(x_pad, W1, Wself1, nidx_flat)
    h2 = _layer(h1, W2, Wself2, nidx_flat)

    subj = triples[:, 0].astype(jnp.int32)
    obj = triples[:, 1].astype(jnp.int32)
    rel = triples[:, 2].astype(jnp.int32)
    rel_perm = jnp.transpose(
        rel_emb.reshape(R, D // 32, 16, 2), (0, 1, 3, 2)).reshape(R, D)
    h2_packed = lax.bitcast_convert_type(
        h2.astype(jnp.bfloat16).reshape(NP, D // 2, 2), jnp.int32)
    scores = _distmult(h2_packed, rel_perm, subj, obj, rel)
    return scores.reshape(B, 1)
